# Initial kernel scaffold; baseline (speedup 1.0000x reference)
import jax
import jax.numpy as jnp
from jax.experimental import pallas as pl


def kernel(x, edge_index, W1l, b1, W1r, W2l, b2, W2r):
    raise NotImplementedError("write your pallas kernel here")


# speedup vs baseline: 5.5430x; 5.5430x over previous
import functools

import jax
import jax.numpy as jnp
from jax import lax
from jax.experimental import pallas as pl
from jax.experimental.pallas import tpu as pltpu
from jax.experimental.pallas import tpu_sc as plsc

N = 10000
E = 320000
NC = 2
NS = 16
K = 128
CH = 158
EPAD = NS * CH * K
NPAD = 10016
RPT = NPAD // NS


@functools.lru_cache(maxsize=None)
def _make_sc_agg(F, with_count):
    mesh = plsc.VectorSubcoreMesh(core_axis_name="c", subcore_axis_name="s",
                                  num_cores=NC, num_subcores=NS)
    outs = [jax.ShapeDtypeStruct((NC, NPAD, F), jnp.float32)]
    scratch = [
        pltpu.VMEM((CH, K), jnp.int32),
        pltpu.VMEM((CH, K), jnp.int32),
        pltpu.VMEM((K, F), jnp.float32),
        pltpu.VMEM((RPT // 2, F), jnp.float32),
        pltpu.VMEM_SHARED((NPAD, F), jnp.float32),
        pltpu.SemaphoreType.DMA,
    ]
    if with_count:
        outs.append(jax.ShapeDtypeStruct((NPAD, 16), jnp.float32))
        scratch += [
            pltpu.VMEM((K, 16), jnp.float32),
            pltpu.VMEM((RPT, 16), jnp.float32),
            pltpu.VMEM_SHARED((NPAD, 16), jnp.float32),
        ]

    def body(vals, gidx, didx, *rest):
        if with_count:
            (out, cnt_out, srcv, dstv, buf, zbuf, acc, sem,
             ones, zcnt, cacc) = rest
        else:
            out, srcv, dstv, buf, zbuf, acc, sem = rest
        c = lax.axis_index("c")
        s = lax.axis_index("s")
        pltpu.sync_copy(gidx.at[c, s], srcv)
        pltpu.sync_copy(didx.at[s], dstv)

        z16 = jnp.zeros((16,), jnp.float32)
        nq = F // 16

        def zrow(r, carry):
            for q in range(nq):
                zbuf[r, pl.ds(q * 16, 16)] = z16
            return carry
        lax.fori_loop(0, RPT // 2, zrow, 0)
        base = s * RPT
        pltpu.sync_copy(zbuf, acc.at[pl.ds(base, RPT // 2)])
        pltpu.sync_copy(zbuf, acc.at[pl.ds(base + RPT // 2, RPT // 2)])

        if with_count:
            one16 = jnp.ones((16,), jnp.float32)

            def onerow(r, carry):
                ones[r, pl.ds(0, 16)] = one16
                return carry
            lax.fori_loop(0, K, onerow, 0)

            def zcrow(r, carry):
                zcnt[r, pl.ds(0, 16)] = z16
                return carry
            lax.fori_loop(0, RPT, zcrow, 0)

            @pl.when(c == 0)
            def _():
                pltpu.sync_copy(zcnt, cacc.at[pl.ds(base, RPT)])

        plsc.subcore_barrier()

        def chunk(j, carry):
            pltpu.async_copy(vals.at[srcv.at[j]], buf, sem).wait()
            pltpu.sync_copy(buf, acc.at[dstv.at[j]], add=True)
            if with_count:
                @pl.when(c == 0)
                def _():
                    pltpu.sync_copy(ones, cacc.at[dstv.at[j]], add=True)
            return carry
        lax.fori_loop(0, CH, chunk, 0)

        plsc.subcore_barrier()
        pltpu.sync_copy(acc.at[pl.ds(base, RPT)], out.at[c, pl.ds(base, RPT)])
        if with_count:
            @pl.when(c == 0)
            def _():
                pltpu.sync_copy(cacc.at[pl.ds(base, RPT)],
                                cnt_out.at[pl.ds(base, RPT)])

    return pl.kernel(body, out_type=tuple(outs), mesh=mesh,
                     scratch_types=scratch,
                     compiler_params=pltpu.CompilerParams(
                         use_tc_tiling_on_sc=False))


def _sc_agg_l1(vals, gidx, didx):
    return _make_sc_agg(64, True)(vals, gidx, didx)


def _sc_agg_l2(vals, gidx, didx):
    out = _make_sc_agg(32, False)(vals, gidx, didx)
    return out[0] if isinstance(out, (tuple, list)) else out


def _tc_layer1(agg1, cnt, x, W1l, b1, W1r, W2l):
    R = 1000

    def body(agg_ref, cnt_ref, x_ref, wl_ref, b_ref, wr_ref, w2_ref,
             h_ref, hw_ref):
        invc = 1.0 / jnp.maximum(cnt_ref[...], 1.0)
        mean = agg_ref[...] * invc
        h = jnp.dot(mean, wl_ref[...], preferred_element_type=jnp.float32)
        h += jnp.dot(x_ref[...], wr_ref[...],
                     preferred_element_type=jnp.float32)
        h = jnp.maximum(h + b_ref[...], 0.0)
        h_ref[...] = h
        hw_ref[...] = jnp.dot(h, w2_ref[...],
                              preferred_element_type=jnp.float32)

    return pl.pallas_call(
        body,
        grid=(N // R,),
        in_specs=[pl.BlockSpec((R, 128), lambda i: (i, 0)),
                  pl.BlockSpec((R, 1), lambda i: (i, 0)),
                  pl.BlockSpec((R, 128), lambda i: (i, 0)),
                  pl.BlockSpec((128, 128), lambda i: (0, 0)),
                  pl.BlockSpec((1, 128), lambda i: (0, 0)),
                  pl.BlockSpec((128, 128), lambda i: (0, 0)),
                  pl.BlockSpec((128, 64), lambda i: (0, 0))],
        out_specs=[pl.BlockSpec((R, 128), lambda i: (i, 0)),
                   pl.BlockSpec((R, 64), lambda i: (i, 0))],
        out_shape=[jax.ShapeDtypeStruct((N, 128), jnp.float32),
                   jax.ShapeDtypeStruct((N, 64), jnp.float32)],
    )(agg1, cnt, x, W1l, b1, W1r, W2l)


def _tc_layer2(agg2, cnt, h, W2r, b2):
    R = 1000

    def body(agg_ref, cnt_ref, h_ref, wr_ref, b_ref, out_ref):
        invc = 1.0 / jnp.maximum(cnt_ref[...], 1.0)
        o = agg_ref[...] * invc
        o += jnp.dot(h_ref[...], wr_ref[...],
                     preferred_element_type=jnp.float32)
        o += b_ref[...]
        m = jnp.max(o, axis=1, keepdims=True)
        e = o - m
        out_ref[...] = e - jnp.log(jnp.sum(jnp.exp(e), axis=1,
                                           keepdims=True))

    return pl.pallas_call(
        body,
        grid=(N // R,),
        in_specs=[pl.BlockSpec((R, 64), lambda i: (i, 0)),
                  pl.BlockSpec((R, 1), lambda i: (i, 0)),
                  pl.BlockSpec((R, 128), lambda i: (i, 0)),
                  pl.BlockSpec((128, 64), lambda i: (0, 0)),
                  pl.BlockSpec((1, 64), lambda i: (0, 0))],
        out_specs=pl.BlockSpec((R, 64), lambda i: (i, 0)),
        out_shape=jax.ShapeDtypeStruct((N, 64), jnp.float32),
    )(agg2, cnt, h, W2r, b2)


def kernel(x, edge_index, W1l, b1, W1r, W2l, b2, W2r):
    src = edge_index[0].astype(jnp.int32)
    dst = edge_index[1].astype(jnp.int32)
    pad = EPAD - E
    srcp = jnp.concatenate([src, jnp.zeros((pad,), jnp.int32)])
    dstp = jnp.concatenate([dst, jnp.full((pad,), N, jnp.int32)])
    didx = dstp.reshape(NS, CH, K)
    s3 = srcp.reshape(NS, CH, K)
    gidx = jnp.stack([s3, s3 + N])

    x2 = jnp.concatenate([x[:, :64], x[:, 64:]], axis=0)
    agg_raw, cnt_raw = _sc_agg_l1(x2, gidx, didx)
    agg1 = agg_raw.transpose(1, 0, 2).reshape(NPAD, 128)[:N]
    cnt = cnt_raw[:N, :1]

    h, hw = _tc_layer1(agg1, cnt, x, W1l, b1.reshape(1, 128), W1r, W2l)

    hw2 = jnp.concatenate([hw[:, :32], hw[:, 32:]], axis=0)
    agg2_raw = _sc_agg_l2(hw2, gidx, didx)
    agg2 = agg2_raw.transpose(1, 0, 2).reshape(NPAD, 64)[:N]

    return _tc_layer2(agg2, cnt, h, W2r, b2.reshape(1, 64))


# speedup vs baseline: 5.8334x; 1.0524x over previous
import functools

import jax
import jax.numpy as jnp
from jax import lax
from jax.experimental import pallas as pl
from jax.experimental.pallas import tpu as pltpu
from jax.experimental.pallas import tpu_sc as plsc

N = 10000
E = 320000
NC = 2
NS = 16
EPAD = 323584
NPAD = 10016
RPT = NPAD // NS


@functools.lru_cache(maxsize=None)
def _make_sc_agg(F, K, with_count):
    CH = EPAD // (NS * K)
    mesh = plsc.VectorSubcoreMesh(core_axis_name="c", subcore_axis_name="s",
                                  num_cores=NC, num_subcores=NS)
    outs = [jax.ShapeDtypeStruct((NC, NPAD, F), jnp.float32)]
    scratch = [
        pltpu.VMEM((CH, K), jnp.int32),
        pltpu.VMEM((CH, K), jnp.int32),
        pltpu.VMEM((K, F), jnp.float32),
        pltpu.VMEM((K, F), jnp.float32),
        pltpu.VMEM((RPT // 2, F), jnp.float32),
        pltpu.VMEM_SHARED((NPAD, F), jnp.float32),
        pltpu.SemaphoreType.DMA,
        pltpu.SemaphoreType.DMA,
    ]
    if with_count:
        outs.append(jax.ShapeDtypeStruct((NPAD, 16), jnp.float32))
        scratch += [
            pltpu.VMEM((K, 16), jnp.float32),
            pltpu.VMEM((RPT, 16), jnp.float32),
            pltpu.VMEM_SHARED((NPAD, 16), jnp.float32),
        ]

    def body(vals, gidx, didx, *rest):
        if with_count:
            (out, cnt_out, srcv, dstv, bufa, bufb, zbuf, acc, sema, semb,
             ones, zcnt, cacc) = rest
        else:
            out, srcv, dstv, bufa, bufb, zbuf, acc, sema, semb = rest
        c = lax.axis_index("c")
        s = lax.axis_index("s")
        pltpu.sync_copy(gidx.at[c, s], srcv)
        vrows = vals
        pltpu.sync_copy(didx.at[s], dstv)

        z16 = jnp.zeros((16,), jnp.float32)
        nq = F // 16

        def zrow(r, carry):
            for q in range(nq):
                zbuf[r, pl.ds(q * 16, 16)] = z16
            return carry
        lax.fori_loop(0, RPT // 2, zrow, 0)
        base = s * RPT
        pltpu.sync_copy(zbuf, acc.at[pl.ds(base, RPT // 2)])
        pltpu.sync_copy(zbuf, acc.at[pl.ds(base + RPT // 2, RPT // 2)])

        if with_count:
            one16 = jnp.ones((16,), jnp.float32)

            def onerow(r, carry):
                ones[r, pl.ds(0, 16)] = one16
                return carry
            lax.fori_loop(0, K, onerow, 0)

            def zcrow(r, carry):
                zcnt[r, pl.ds(0, 16)] = z16
                return carry
            lax.fori_loop(0, RPT, zcrow, 0)

            @pl.when(c == 0)
            def _():
                pltpu.sync_copy(zcnt, cacc.at[pl.ds(base, RPT)])

        plsc.subcore_barrier()

        def scat(j, buf):
            pltpu.sync_copy(buf, acc.at[dstv.at[j]], add=True)
            if with_count:
                @pl.when(c == 0)
                def _():
                    pltpu.sync_copy(ones, cacc.at[dstv.at[j]], add=True)

        def pair(jp, carry):
            jj = jp * 2
            da = pltpu.async_copy(vrows.at[srcv.at[jj]], bufa, sema)
            db = pltpu.async_copy(vrows.at[srcv.at[jj + 1]], bufb, semb)
            da.wait()
            scat(jj, bufa)
            db.wait()
            scat(jj + 1, bufb)
            return carry
        lax.fori_loop(0, CH // 2, pair, 0)

        plsc.subcore_barrier()
        pltpu.sync_copy(acc.at[pl.ds(base, RPT)], out.at[c, pl.ds(base, RPT)])
        if with_count:
            @pl.when(c == 0)
            def _():
                pltpu.sync_copy(cacc.at[pl.ds(base, RPT)],
                                cnt_out.at[pl.ds(base, RPT)])

    return pl.kernel(body, out_type=tuple(outs), mesh=mesh,
                     scratch_types=scratch,
                     compiler_params=pltpu.CompilerParams(
                         use_tc_tiling_on_sc=False))


def _edge_layout(srcp, dstp, K):
    CH = EPAD // (NS * K)
    didx = dstp.reshape(NS, CH, K)
    s3 = srcp.reshape(NS, CH, K)
    gidx = jnp.stack([s3, s3 + N])
    return gidx, didx


def _sc_agg_l1(vals, srcp, dstp):
    gidx, didx = _edge_layout(srcp, dstp, 64)
    return _make_sc_agg(64, 64, True)(vals, gidx, didx)


def _sc_agg_l2(vals, srcp, dstp):
    gidx, didx = _edge_layout(srcp, dstp, 128)
    out = _make_sc_agg(32, 128, False)(vals, gidx, didx)
    return out[0] if isinstance(out, (tuple, list)) else out


def _tc_layer1(agg1, cnt, x, W1l, b1, W1r, W2l):
    R = 1000

    def body(agg_ref, cnt_ref, x_ref, wl_ref, b_ref, wr_ref, w2_ref,
             h_ref, hw_ref):
        invc = 1.0 / jnp.maximum(cnt_ref[...], 1.0)
        mean = agg_ref[...] * invc
        h = jnp.dot(mean, wl_ref[...], preferred_element_type=jnp.float32)
        h += jnp.dot(x_ref[...], wr_ref[...],
                     preferred_element_type=jnp.float32)
        h = jnp.maximum(h + b_ref[...], 0.0)
        h_ref[...] = h
        hw_ref[...] = jnp.dot(h, w2_ref[...],
                              preferred_element_type=jnp.float32)

    return pl.pallas_call(
        body,
        grid=(N // R,),
        in_specs=[pl.BlockSpec((R, 128), lambda i: (i, 0)),
                  pl.BlockSpec((R, 1), lambda i: (i, 0)),
                  pl.BlockSpec((R, 128), lambda i: (i, 0)),
                  pl.BlockSpec((128, 128), lambda i: (0, 0)),
                  pl.BlockSpec((1, 128), lambda i: (0, 0)),
                  pl.BlockSpec((128, 128), lambda i: (0, 0)),
                  pl.BlockSpec((128, 64), lambda i: (0, 0))],
        out_specs=[pl.BlockSpec((R, 128), lambda i: (i, 0)),
                   pl.BlockSpec((R, 64), lambda i: (i, 0))],
        out_shape=[jax.ShapeDtypeStruct((N, 128), jnp.float32),
                   jax.ShapeDtypeStruct((N, 64), jnp.float32)],
    )(agg1, cnt, x, W1l, b1, W1r, W2l)


def _tc_layer2(agg2, cnt, h, W2r, b2):
    R = 1000

    def body(agg_ref, cnt_ref, h_ref, wr_ref, b_ref, out_ref):
        invc = 1.0 / jnp.maximum(cnt_ref[...], 1.0)
        o = agg_ref[...] * invc
        o += jnp.dot(h_ref[...], wr_ref[...],
                     preferred_element_type=jnp.float32)
        o += b_ref[...]
        m = jnp.max(o, axis=1, keepdims=True)
        e = o - m
        out_ref[...] = e - jnp.log(jnp.sum(jnp.exp(e), axis=1,
                                           keepdims=True))

    return pl.pallas_call(
        body,
        grid=(N // R,),
        in_specs=[pl.BlockSpec((R, 64), lambda i: (i, 0)),
                  pl.BlockSpec((R, 1), lambda i: (i, 0)),
                  pl.BlockSpec((R, 128), lambda i: (i, 0)),
                  pl.BlockSpec((128, 64), lambda i: (0, 0)),
                  pl.BlockSpec((1, 64), lambda i: (0, 0))],
        out_specs=pl.BlockSpec((R, 64), lambda i: (i, 0)),
        out_shape=jax.ShapeDtypeStruct((N, 64), jnp.float32),
    )(agg2, cnt, h, W2r, b2)


def kernel(x, edge_index, W1l, b1, W1r, W2l, b2, W2r):
    src = edge_index[0].astype(jnp.int32)
    dst = edge_index[1].astype(jnp.int32)
    pad = EPAD - E
    srcp = jnp.concatenate([src, jnp.zeros((pad,), jnp.int32)])
    dstp = jnp.concatenate([dst, jnp.full((pad,), N, jnp.int32)])

    x2 = jnp.concatenate([x[:, :64], x[:, 64:]], axis=0)
    agg_raw, cnt_raw = _sc_agg_l1(x2, srcp, dstp)
    agg1 = agg_raw.transpose(1, 0, 2).reshape(NPAD, 128)[:N]
    cnt = cnt_raw[:N, :1]

    h, hw = _tc_layer1(agg1, cnt, x, W1l, b1.reshape(1, 128), W1r, W2l)

    hw2 = jnp.concatenate([hw[:, :32], hw[:, 32:]], axis=0)
    agg2_raw = _sc_agg_l2(hw2, srcp, dstp)
    agg2 = agg2_raw.transpose(1, 0, 2).reshape(NPAD, 64)[:N]

    return _tc_layer2(agg2, cnt, h, W2r, b2.reshape(1, 64))


# speedup vs baseline: 6.1234x; 1.0497x over previous
import functools

import jax
import jax.numpy as jnp
from jax import lax
from jax.experimental import pallas as pl
from jax.experimental.pallas import tpu as pltpu
from jax.experimental.pallas import tpu_sc as plsc

N = 10000
E = 320000
NC = 2
NS = 16
EPAD = 323584
NPAD = 10016
RPT = NPAD // NS


@functools.lru_cache(maxsize=None)
def _make_sc_agg(F, K, with_count):
    CH = EPAD // (NS * K)
    mesh = plsc.VectorSubcoreMesh(core_axis_name="c", subcore_axis_name="s",
                                  num_cores=NC, num_subcores=NS)
    outs = [jax.ShapeDtypeStruct((NC, NPAD, F), jnp.float32)]
    scratch = [
        pltpu.VMEM((CH, K), jnp.int32),
        pltpu.VMEM((CH, K), jnp.int32),
        pltpu.VMEM((K, F), jnp.float32),
        pltpu.VMEM((K, F), jnp.float32),
        pltpu.VMEM((RPT // 2, F), jnp.float32),
        pltpu.VMEM_SHARED((NPAD, F), jnp.float32),
        pltpu.SemaphoreType.DMA,
        pltpu.SemaphoreType.DMA,
        pltpu.SemaphoreType.DMA,
        pltpu.SemaphoreType.DMA,
    ]
    if with_count:
        outs.append(jax.ShapeDtypeStruct((NPAD, 16), jnp.float32))
        scratch += [
            pltpu.VMEM((K, 16), jnp.float32),
            pltpu.VMEM((RPT, 16), jnp.float32),
            pltpu.VMEM_SHARED((NPAD, 16), jnp.float32),
        ]

    def body(vals, gidx, didx, *rest):
        if with_count:
            (out, cnt_out, srcv, dstv, bufa, bufb, zbuf, acc, sema, semb,
             ssema, ssemb, ones, zcnt, cacc) = rest
        else:
            (out, srcv, dstv, bufa, bufb, zbuf, acc, sema, semb,
             ssema, ssemb) = rest
        c = lax.axis_index("c")
        s = lax.axis_index("s")
        pltpu.sync_copy(gidx.at[c, s], srcv)
        vrows = vals
        pltpu.sync_copy(didx.at[s], dstv)

        z16 = jnp.zeros((16,), jnp.float32)
        nq = F // 16

        def zrow(r, carry):
            for q in range(nq):
                zbuf[r, pl.ds(q * 16, 16)] = z16
            return carry
        lax.fori_loop(0, RPT // 2, zrow, 0)
        base = s * RPT
        pltpu.sync_copy(zbuf, acc.at[pl.ds(base, RPT // 2)])
        pltpu.sync_copy(zbuf, acc.at[pl.ds(base + RPT // 2, RPT // 2)])

        if with_count:
            one16 = jnp.ones((16,), jnp.float32)

            def onerow(r, carry):
                ones[r, pl.ds(0, 16)] = one16
                return carry
            lax.fori_loop(0, K, onerow, 0)

            def zcrow(r, carry):
                zcnt[r, pl.ds(0, 16)] = z16
                return carry
            lax.fori_loop(0, RPT, zcrow, 0)

            @pl.when(c == 0)
            def _():
                pltpu.sync_copy(zcnt, cacc.at[pl.ds(base, RPT)])

        plsc.subcore_barrier()

        def scat(j, buf, ssem):
            d = pltpu.async_copy(buf, acc.at[dstv.at[j]], ssem, add=True)
            if with_count:
                @pl.when(c == 0)
                def _():
                    pltpu.sync_copy(ones, cacc.at[dstv.at[j]], add=True)
            return d

        def pair(jp, carry):
            jj = jp * 2

            @pl.when(jp > 0)
            def _():
                pltpu.make_async_copy(bufa, acc.at[dstv.at[jj - 2]],
                                      ssema).wait()
                pltpu.make_async_copy(bufb, acc.at[dstv.at[jj - 1]],
                                      ssemb).wait()
            da = pltpu.async_copy(vrows.at[srcv.at[jj]], bufa, sema)
            db = pltpu.async_copy(vrows.at[srcv.at[jj + 1]], bufb, semb)
            da.wait()
            scat(jj, bufa, ssema)
            db.wait()
            scat(jj + 1, bufb, ssemb)
            return carry
        lax.fori_loop(0, CH // 2, pair, 0)
        pltpu.make_async_copy(bufa, acc.at[dstv.at[CH - 2]], ssema).wait()
        pltpu.make_async_copy(bufb, acc.at[dstv.at[CH - 1]], ssemb).wait()

        plsc.subcore_barrier()
        pltpu.sync_copy(acc.at[pl.ds(base, RPT)], out.at[c, pl.ds(base, RPT)])
        if with_count:
            @pl.when(c == 0)
            def _():
                pltpu.sync_copy(cacc.at[pl.ds(base, RPT)],
                                cnt_out.at[pl.ds(base, RPT)])

    return pl.kernel(body, out_type=tuple(outs), mesh=mesh,
                     scratch_types=scratch,
                     compiler_params=pltpu.CompilerParams(
                         use_tc_tiling_on_sc=False))


def _edge_layout(srcp, dstp, K):
    CH = EPAD // (NS * K)
    didx = dstp.reshape(NS, CH, K)
    s3 = srcp.reshape(NS, CH, K)
    gidx = jnp.stack([s3, s3 + N])
    return gidx, didx


def _sc_agg_l1(vals, srcp, dstp):
    gidx, didx = _edge_layout(srcp, dstp, 64)
    return _make_sc_agg(64, 64, True)(vals, gidx, didx)


def _sc_agg_l2(vals, srcp, dstp):
    gidx, didx = _edge_layout(srcp, dstp, 128)
    out = _make_sc_agg(32, 128, False)(vals, gidx, didx)
    return out[0] if isinstance(out, (tuple, list)) else out


def _tc_layer1(agg1, cnt, x, W1l, b1, W1r, W2l):
    R = 1000

    def body(agg_ref, cnt_ref, x_ref, wl_ref, b_ref, wr_ref, w2_ref,
             h_ref, hw_ref):
        invc = 1.0 / jnp.maximum(cnt_ref[...], 1.0)
        mean = agg_ref[...] * invc
        h = jnp.dot(mean, wl_ref[...], preferred_element_type=jnp.float32)
        h += jnp.dot(x_ref[...], wr_ref[...],
                     preferred_element_type=jnp.float32)
        h = jnp.maximum(h + b_ref[...], 0.0)
        h_ref[...] = h
        hw_ref[...] = jnp.dot(h, w2_ref[...],
                              preferred_element_type=jnp.float32)

    return pl.pallas_call(
        body,
        grid=(N // R,),
        in_specs=[pl.BlockSpec((R, 128), lambda i: (i, 0)),
                  pl.BlockSpec((R, 1), lambda i: (i, 0)),
                  pl.BlockSpec((R, 128), lambda i: (i, 0)),
                  pl.BlockSpec((128, 128), lambda i: (0, 0)),
                  pl.BlockSpec((1, 128), lambda i: (0, 0)),
                  pl.BlockSpec((128, 128), lambda i: (0, 0)),
                  pl.BlockSpec((128, 64), lambda i: (0, 0))],
        out_specs=[pl.BlockSpec((R, 128), lambda i: (i, 0)),
                   pl.BlockSpec((R, 64), lambda i: (i, 0))],
        out_shape=[jax.ShapeDtypeStruct((N, 128), jnp.float32),
                   jax.ShapeDtypeStruct((N, 64), jnp.float32)],
    )(agg1, cnt, x, W1l, b1, W1r, W2l)


def _tc_layer2(agg2, cnt, h, W2r, b2):
    R = 1000

    def body(agg_ref, cnt_ref, h_ref, wr_ref, b_ref, out_ref):
        invc = 1.0 / jnp.maximum(cnt_ref[...], 1.0)
        o = agg_ref[...] * invc
        o += jnp.dot(h_ref[...], wr_ref[...],
                     preferred_element_type=jnp.float32)
        o += b_ref[...]
        m = jnp.max(o, axis=1, keepdims=True)
        e = o - m
        out_ref[...] = e - jnp.log(jnp.sum(jnp.exp(e), axis=1,
                                           keepdims=True))

    return pl.pallas_call(
        body,
        grid=(N // R,),
        in_specs=[pl.BlockSpec((R, 64), lambda i: (i, 0)),
                  pl.BlockSpec((R, 1), lambda i: (i, 0)),
                  pl.BlockSpec((R, 128), lambda i: (i, 0)),
                  pl.BlockSpec((128, 64), lambda i: (0, 0)),
                  pl.BlockSpec((1, 64), lambda i: (0, 0))],
        out_specs=pl.BlockSpec((R, 64), lambda i: (i, 0)),
        out_shape=jax.ShapeDtypeStruct((N, 64), jnp.float32),
    )(agg2, cnt, h, W2r, b2)


def kernel(x, edge_index, W1l, b1, W1r, W2l, b2, W2r):
    src = edge_index[0].astype(jnp.int32)
    dst = edge_index[1].astype(jnp.int32)
    pad = EPAD - E
    srcp = jnp.concatenate([src, jnp.zeros((pad,), jnp.int32)])
    dstp = jnp.concatenate([dst, jnp.full((pad,), N, jnp.int32)])

    x2 = jnp.concatenate([x[:, :64], x[:, 64:]], axis=0)
    agg_raw, cnt_raw = _sc_agg_l1(x2, srcp, dstp)
    agg1 = agg_raw.transpose(1, 0, 2).reshape(NPAD, 128)[:N]
    cnt = cnt_raw[:N, :1]

    h, hw = _tc_layer1(agg1, cnt, x, W1l, b1.reshape(1, 128), W1r, W2l)

    hw2 = jnp.concatenate([hw[:, :32], hw[:, 32:]], axis=0)
    agg2_raw = _sc_agg_l2(hw2, srcp, dstp)
    agg2 = agg2_raw.transpose(1, 0, 2).reshape(NPAD, 64)[:N]

    return _tc_layer2(agg2, cnt, h, W2r, b2.reshape(1, 64))
